# Initial kernel scaffold; baseline (speedup 1.0000x reference)
import jax
import jax.numpy as jnp
from jax.experimental import pallas as pl


def kernel(x, edge_index, W_LP1, W_HP1, b1, a1, W_LP2, W_HP2, b2, a2):
    raise NotImplementedError("write your pallas kernel here")


# speedup vs baseline: 10.9755x; 10.9755x over previous
import functools

import jax
import jax.numpy as jnp
from jax import lax
from jax.experimental import pallas as pl
from jax.experimental.pallas import tpu as pltpu
from jax.experimental.pallas import tpu_sc as plsc

N_SC = 2
N_SUB = 16
LANES = 128
DUMMY = 16


def _sc_mesh():
    return plsc.VectorSubcoreMesh(core_axis_name="c", subcore_axis_name="s")


def _make_deg_kernel(n_nodes, ep_rows):
    rows_per_tile = ep_rows // (N_SC * N_SUB)
    acc_n = n_nodes + DUMMY

    @functools.partial(
        pl.kernel,
        out_type=jax.ShapeDtypeStruct((N_SC, n_nodes, LANES), jnp.float32),
        mesh=_sc_mesh(),
        scratch_types=[
            pltpu.VMEM((rows_per_tile, LANES), jnp.int32),
            pltpu.VMEM((LANES, LANES), jnp.float32),
            pltpu.VMEM_SHARED((acc_n, LANES), jnp.float32),
        ],
    )
    def deg_kernel(rowsp_hbm, zeros_hbm, out_hbm, idx_v, ones_v, acc):
        c = lax.axis_index("c")
        s = lax.axis_index("s")

        @pl.loop(0, LANES)
        def _(r):
            for j in range(LANES // 16):
                ones_v[r, pl.ds(j * 16, 16)] = jnp.full((16,), 1.0, jnp.float32)

        zi_rows = acc_n // N_SUB // 8 * 8
        zi_last = acc_n - (N_SUB - 1) * zi_rows

        @pl.when(s < N_SUB - 1)
        def _():
            pltpu.sync_copy(zeros_hbm.at[pl.ds(s * zi_rows, zi_rows)],
                            acc.at[pl.ds(s * zi_rows, zi_rows)])

        @pl.when(s == N_SUB - 1)
        def _():
            pltpu.sync_copy(zeros_hbm.at[pl.ds((N_SUB - 1) * zi_rows, zi_last)],
                            acc.at[pl.ds((N_SUB - 1) * zi_rows, zi_last)])

        base = (c * N_SUB + s) * rows_per_tile
        pltpu.sync_copy(rowsp_hbm.at[pl.ds(base, rows_per_tile)], idx_v)
        plsc.subcore_barrier()

        @pl.loop(0, rows_per_tile)
        def _(r):
            pltpu.sync_copy(ones_v, acc.at[idx_v.at[r]], add=True)

        plsc.subcore_barrier()

        wb_rows = n_nodes // N_SUB // 8 * 8
        wb_last = n_nodes - (N_SUB - 1) * wb_rows

        @pl.when(s < N_SUB - 1)
        def _():
            pltpu.sync_copy(acc.at[pl.ds(s * wb_rows, wb_rows)],
                            out_hbm.at[c].at[pl.ds(s * wb_rows, wb_rows)])

        @pl.when(s == N_SUB - 1)
        def _():
            pltpu.sync_copy(acc.at[pl.ds((N_SUB - 1) * wb_rows, wb_last)],
                            out_hbm.at[c].at[pl.ds((N_SUB - 1) * wb_rows, wb_last)])

    return deg_kernel


def _make_spmm_kernel(n_nodes, ep_rows, n_blocks):
    blocks_per_core = n_blocks // N_SC
    rows_per_tile = ep_rows // N_SUB
    wb_rows = (n_nodes // N_SUB) // 8 * 8
    wb_last = n_nodes - (N_SUB - 1) * wb_rows
    acc_n = n_nodes + DUMMY

    @functools.partial(
        pl.kernel,
        out_type=jax.ShapeDtypeStruct((n_blocks * n_nodes, LANES), jnp.float32),
        mesh=_sc_mesh(),
        scratch_types=[
            pltpu.VMEM((rows_per_tile, LANES), jnp.int32),
            pltpu.VMEM((rows_per_tile, LANES), jnp.int32),
            pltpu.VMEM((LANES, LANES), jnp.float32),
            pltpu.VMEM_SHARED((acc_n, LANES), jnp.float32),
        ],
    )
    def spmm_kernel(xp_hbm, rowsp_hbm, colsp_hbm, z_hbm, idx_c, idx_r, gath, acc):
        c = lax.axis_index("c")
        s = lax.axis_index("s")

        for bl in range(blocks_per_core):
            blk = c * blocks_per_core + bl
            @pl.when(s < N_SUB - 1)
            def _():
                pltpu.sync_copy(
                    xp_hbm.at[pl.ds(blk * n_nodes + s * wb_rows, wb_rows)],
                    acc.at[pl.ds(s * wb_rows, wb_rows)],
                )

            @pl.when(s == N_SUB - 1)
            def _():
                pltpu.sync_copy(
                    xp_hbm.at[pl.ds(blk * n_nodes + (N_SUB - 1) * wb_rows, wb_last)],
                    acc.at[pl.ds((N_SUB - 1) * wb_rows, wb_last)],
                )
            pltpu.sync_copy(colsp_hbm.at[pl.ds(s * rows_per_tile, rows_per_tile)], idx_c)
            pltpu.sync_copy(rowsp_hbm.at[pl.ds(s * rows_per_tile, rows_per_tile)], idx_r)
            off = blk * n_nodes

            @pl.loop(0, rows_per_tile)
            def _(r):
                for j in range(LANES // 16):
                    sl = (r, pl.ds(j * 16, 16))
                    idx_c[sl] = idx_c[sl] + off

            plsc.subcore_barrier()

            @pl.loop(0, rows_per_tile)
            def _(r):
                pltpu.sync_copy(xp_hbm.at[idx_c.at[r]], gath)
                pltpu.sync_copy(gath, acc.at[idx_r.at[r]], add=True)

            plsc.subcore_barrier()

            @pl.when(s < N_SUB - 1)
            def _():
                pltpu.sync_copy(
                    acc.at[pl.ds(s * wb_rows, wb_rows)],
                    z_hbm.at[pl.ds(blk * n_nodes + s * wb_rows, wb_rows)],
                )

            @pl.when(s == N_SUB - 1)
            def _():
                pltpu.sync_copy(
                    acc.at[pl.ds((N_SUB - 1) * wb_rows, wb_last)],
                    z_hbm.at[pl.ds(blk * n_nodes + (N_SUB - 1) * wb_rows, wb_last)],
                )

    return spmm_kernel


def _dis_scale_kernel(degp_ref, x_ref, dis_ref, xp_ref):
    deg = degp_ref[0][:, 0:1] + degp_ref[1][:, 0:1] + 1.0
    dis = lax.rsqrt(deg)
    dis_ref[...] = dis
    xp_ref[0] = x_ref[...] * dis


def _layer1_kernel(z_ref, x_ref, dis_ref, wl_ref, wh_ref, b_ref, a_ref,
                   h_ref, xp2_ref):
    dis = dis_ref[...]
    sh = jnp.concatenate([z_ref[0], z_ref[1]], axis=1) * dis
    alpha = 1.0 / (1.0 + jnp.exp(-a_ref[0, 0]))
    m1 = jnp.dot(sh, wl_ref[...], preferred_element_type=jnp.float32)
    m2 = jnp.dot(x_ref[...] - sh, wh_ref[...], preferred_element_type=jnp.float32)
    h = jnp.maximum(alpha * m1 + (1.0 - alpha) * m2 + b_ref[...], 0.0)
    h_ref[...] = h
    for j in range(4):
        xp2_ref[j] = h[:, j * LANES:(j + 1) * LANES] * dis


def _layer2_kernel(z_ref, h_ref, dis_ref, wl_ref, wh_ref, b_ref, a_ref, out_ref):
    dis = dis_ref[...]
    sh = jnp.concatenate([z_ref[j] for j in range(4)], axis=1) * dis
    alpha = 1.0 / (1.0 + jnp.exp(-a_ref[0, 0]))
    m1 = jnp.dot(sh, wl_ref[...], preferred_element_type=jnp.float32)
    m2 = jnp.dot(h_ref[...] - sh, wh_ref[...], preferred_element_type=jnp.float32)
    out_ref[...] = alpha * m1 + (1.0 - alpha) * m2 + b_ref[...]


def kernel(x, edge_index, W_LP1, W_HP1, b1, a1, W_LP2, W_HP2, b2, a2):
    n, f_in = x.shape
    hid = W_LP1.shape[1]
    c_out = W_LP2.shape[1]
    e = edge_index.shape[1]
    nb1 = f_in // LANES
    nb2 = hid // LANES
    rb = 1000
    n_rb = n // rb

    ep_rows = -(-e // (N_SUB * LANES * 2)) * (N_SUB * 2)
    pad = ep_rows * LANES - e
    rows = edge_index[0]
    cols = edge_index[1]
    pad_ix = lax.iota(jnp.int32, pad)
    rowsp = jnp.concatenate([rows, n + (pad_ix % DUMMY)]).reshape(ep_rows, LANES)
    colsp = jnp.concatenate([cols, pad_ix % n]).reshape(ep_rows, LANES)
    zeros_wide = jnp.zeros((n + DUMMY, LANES), jnp.float32)

    degp = _make_deg_kernel(n, ep_rows)(rowsp, zeros_wide)

    dis, xp1 = pl.pallas_call(
        _dis_scale_kernel,
        grid=(n_rb, nb1),
        in_specs=[
            pl.BlockSpec((N_SC, rb, LANES), lambda i, b: (0, i, 0)),
            pl.BlockSpec((rb, LANES), lambda i, b: (i, b)),
        ],
        out_specs=[
            pl.BlockSpec((rb, 1), lambda i, b: (i, 0)),
            pl.BlockSpec((1, rb, LANES), lambda i, b: (b, i, 0)),
        ],
        out_shape=[
            jax.ShapeDtypeStruct((n, 1), jnp.float32),
            jax.ShapeDtypeStruct((nb1, n, LANES), jnp.float32),
        ],
    )(degp, x)

    z1 = _make_spmm_kernel(n, ep_rows, nb1)(
        xp1.reshape(nb1 * n, LANES), rowsp, colsp)

    h, xp2 = pl.pallas_call(
        _layer1_kernel,
        grid=(n_rb,),
        in_specs=[
            pl.BlockSpec((nb1, rb, LANES), lambda i: (0, i, 0)),
            pl.BlockSpec((rb, f_in), lambda i: (i, 0)),
            pl.BlockSpec((rb, 1), lambda i: (i, 0)),
            pl.BlockSpec((f_in, hid), lambda i: (0, 0)),
            pl.BlockSpec((f_in, hid), lambda i: (0, 0)),
            pl.BlockSpec((hid,), lambda i: (0,)),
            pl.BlockSpec((1, 1), lambda i: (0, 0)),
        ],
        out_specs=[
            pl.BlockSpec((rb, hid), lambda i: (i, 0)),
            pl.BlockSpec((nb2, rb, LANES), lambda i: (0, i, 0)),
        ],
        out_shape=[
            jax.ShapeDtypeStruct((n, hid), jnp.float32),
            jax.ShapeDtypeStruct((nb2, n, LANES), jnp.float32),
        ],
    )(z1.reshape(nb1, n, LANES), x, dis, W_LP1, W_HP1, b1, a1.reshape(1, 1))

    z2 = _make_spmm_kernel(n, ep_rows, nb2)(
        xp2.reshape(nb2 * n, LANES), rowsp, colsp)

    out = pl.pallas_call(
        _layer2_kernel,
        grid=(n_rb,),
        in_specs=[
            pl.BlockSpec((nb2, rb, LANES), lambda i: (0, i, 0)),
            pl.BlockSpec((rb, hid), lambda i: (i, 0)),
            pl.BlockSpec((rb, 1), lambda i: (i, 0)),
            pl.BlockSpec((hid, c_out), lambda i: (0, 0)),
            pl.BlockSpec((hid, c_out), lambda i: (0, 0)),
            pl.BlockSpec((c_out,), lambda i: (0,)),
            pl.BlockSpec((1, 1), lambda i: (0, 0)),
        ],
        out_specs=pl.BlockSpec((rb, c_out), lambda i: (i, 0)),
        out_shape=jax.ShapeDtypeStruct((n, c_out), jnp.float32),
    )(z2.reshape(nb2, n, LANES), h, dis, W_LP2, W_HP2, b2, a2.reshape(1, 1))

    return out


# speedup vs baseline: 15.6739x; 1.4281x over previous
import functools

import jax
import jax.numpy as jnp
from jax import lax
from jax.experimental import pallas as pl
from jax.experimental.pallas import tpu as pltpu
from jax.experimental.pallas import tpu_sc as plsc

N_SC = 2
N_SUB = 16
LANES = 128
DUMMY = 16


def _sc_mesh():
    return plsc.VectorSubcoreMesh(core_axis_name="c", subcore_axis_name="s")


def _make_deg_kernel(n_nodes, ep_rows):
    rows_per_tile = ep_rows // (N_SC * N_SUB)
    acc_n = n_nodes + DUMMY

    @functools.partial(
        pl.kernel,
        out_type=jax.ShapeDtypeStruct((N_SC, n_nodes, LANES), jnp.float32),
        mesh=_sc_mesh(),
        scratch_types=[
            pltpu.VMEM((rows_per_tile, LANES), jnp.int32),
            pltpu.VMEM((LANES, LANES), jnp.float32),
            pltpu.VMEM_SHARED((acc_n, LANES), jnp.float32),
        ],
    )
    def deg_kernel(rowsp_hbm, zeros_hbm, out_hbm, idx_v, ones_v, acc):
        c = lax.axis_index("c")
        s = lax.axis_index("s")

        @pl.loop(0, LANES)
        def _(r):
            for j in range(LANES // 16):
                ones_v[r, pl.ds(j * 16, 16)] = jnp.full((16,), 1.0, jnp.float32)

        zi_rows = acc_n // N_SUB // 8 * 8
        zi_last = acc_n - (N_SUB - 1) * zi_rows

        @pl.when(s < N_SUB - 1)
        def _():
            pltpu.sync_copy(zeros_hbm.at[pl.ds(s * zi_rows, zi_rows)],
                            acc.at[pl.ds(s * zi_rows, zi_rows)])

        @pl.when(s == N_SUB - 1)
        def _():
            pltpu.sync_copy(zeros_hbm.at[pl.ds((N_SUB - 1) * zi_rows, zi_last)],
                            acc.at[pl.ds((N_SUB - 1) * zi_rows, zi_last)])

        base = (c * N_SUB + s) * rows_per_tile
        pltpu.sync_copy(rowsp_hbm.at[pl.ds(base, rows_per_tile)], idx_v)
        plsc.subcore_barrier()

        @pl.loop(0, rows_per_tile)
        def _(r):
            pltpu.sync_copy(ones_v, acc.at[idx_v.at[r]], add=True)

        plsc.subcore_barrier()

        wb_rows = n_nodes // N_SUB // 8 * 8
        wb_last = n_nodes - (N_SUB - 1) * wb_rows

        @pl.when(s < N_SUB - 1)
        def _():
            pltpu.sync_copy(acc.at[pl.ds(s * wb_rows, wb_rows)],
                            out_hbm.at[c].at[pl.ds(s * wb_rows, wb_rows)])

        @pl.when(s == N_SUB - 1)
        def _():
            pltpu.sync_copy(acc.at[pl.ds((N_SUB - 1) * wb_rows, wb_last)],
                            out_hbm.at[c].at[pl.ds((N_SUB - 1) * wb_rows, wb_last)])

    return deg_kernel


def _make_spmm_kernel(n_nodes, ep_rows, n_blocks):
    blocks_per_core = n_blocks // N_SC
    rows_per_tile = ep_rows // N_SUB
    wb_rows = (n_nodes // N_SUB) // 8 * 8
    wb_last = n_nodes - (N_SUB - 1) * wb_rows
    acc_n = n_nodes + DUMMY

    nbuf = 2
    grp_rows = 40
    n_grp = rows_per_tile // grp_rows

    @functools.partial(
        pl.kernel,
        out_type=jax.ShapeDtypeStruct((n_blocks * n_nodes, LANES), jnp.float32),
        mesh=_sc_mesh(),
        scratch_types=[
            pltpu.VMEM((grp_rows, LANES), jnp.int32),
            pltpu.VMEM((grp_rows, LANES), jnp.int32),
        ] + [pltpu.VMEM((LANES, LANES), jnp.float32) for _ in range(nbuf)]
          + [pltpu.VMEM_SHARED((acc_n, LANES), jnp.float32)]
          + [pltpu.SemaphoreType.DMA for _ in range(nbuf)],
    )
    def spmm_kernel(xp_hbm, rowsp_hbm, colsp_hbm, z_hbm, idx_c, idx_r,
                    gb0, gb1, acc, sm0, sm1):
        gb = [gb0, gb1]
        sm = [sm0, sm1]
        c = lax.axis_index("c")
        s = lax.axis_index("s")

        for bl in range(blocks_per_core):
            blk = c * blocks_per_core + bl
            @pl.when(s < N_SUB - 1)
            def _():
                pltpu.sync_copy(
                    xp_hbm.at[pl.ds(blk * n_nodes + s * wb_rows, wb_rows)],
                    acc.at[pl.ds(s * wb_rows, wb_rows)],
                )

            @pl.when(s == N_SUB - 1)
            def _():
                pltpu.sync_copy(
                    xp_hbm.at[pl.ds(blk * n_nodes + (N_SUB - 1) * wb_rows, wb_last)],
                    acc.at[pl.ds((N_SUB - 1) * wb_rows, wb_last)],
                )
            off = blk * n_nodes
            plsc.subcore_barrier()

            @pl.loop(0, n_grp)
            def _(g):
                ebase = s * rows_per_tile + g * grp_rows
                pltpu.sync_copy(colsp_hbm.at[pl.ds(ebase, grp_rows)], idx_c)
                pltpu.sync_copy(rowsp_hbm.at[pl.ds(ebase, grp_rows)], idx_r)

                @pl.loop(0, grp_rows)
                def _(r):
                    for j in range(LANES // 16):
                        sl = (r, pl.ds(j * 16, 16))
                        idx_c[sl] = idx_c[sl] + off

                for k in range(nbuf):
                    pltpu.async_copy(xp_hbm.at[idx_c.at[k]], gb[k], sm[k])

                @pl.loop(0, grp_rows - nbuf, step=nbuf)
                def _(r):
                    for k in range(nbuf):
                        pltpu.make_async_copy(
                            xp_hbm.at[idx_c.at[r + k]], gb[k], sm[k]).wait()
                        pltpu.sync_copy(gb[k], acc.at[idx_r.at[r + k]], add=True)
                        pltpu.async_copy(
                            xp_hbm.at[idx_c.at[r + nbuf + k]], gb[k], sm[k])

                for k in range(nbuf):
                    rr = grp_rows - nbuf + k
                    pltpu.make_async_copy(
                        xp_hbm.at[idx_c.at[rr]], gb[k], sm[k]).wait()
                    pltpu.sync_copy(gb[k], acc.at[idx_r.at[rr]], add=True)

            plsc.subcore_barrier()

            @pl.when(s < N_SUB - 1)
            def _():
                pltpu.sync_copy(
                    acc.at[pl.ds(s * wb_rows, wb_rows)],
                    z_hbm.at[pl.ds(blk * n_nodes + s * wb_rows, wb_rows)],
                )

            @pl.when(s == N_SUB - 1)
            def _():
                pltpu.sync_copy(
                    acc.at[pl.ds((N_SUB - 1) * wb_rows, wb_last)],
                    z_hbm.at[pl.ds(blk * n_nodes + (N_SUB - 1) * wb_rows, wb_last)],
                )

    return spmm_kernel


def _dis_scale_kernel(degp_ref, x_ref, dis_ref, xp_ref):
    deg = degp_ref[0][:, 0:1] + degp_ref[1][:, 0:1] + 1.0
    dis = lax.rsqrt(deg)
    dis_ref[...] = dis
    xp_ref[0] = x_ref[...] * dis


def _layer1_kernel(z_ref, x_ref, dis_ref, wl_ref, wh_ref, b_ref, a_ref,
                   h_ref, xp2_ref):
    dis = dis_ref[...]
    sh = jnp.concatenate([z_ref[0], z_ref[1]], axis=1) * dis
    alpha = 1.0 / (1.0 + jnp.exp(-a_ref[0, 0]))
    m1 = jnp.dot(sh, wl_ref[...], preferred_element_type=jnp.float32)
    m2 = jnp.dot(x_ref[...] - sh, wh_ref[...], preferred_element_type=jnp.float32)
    h = jnp.maximum(alpha * m1 + (1.0 - alpha) * m2 + b_ref[...], 0.0)
    h_ref[...] = h
    for j in range(4):
        xp2_ref[j] = h[:, j * LANES:(j + 1) * LANES] * dis


def _layer2_kernel(z_ref, h_ref, dis_ref, wl_ref, wh_ref, b_ref, a_ref, out_ref):
    dis = dis_ref[...]
    sh = jnp.concatenate([z_ref[j] for j in range(4)], axis=1) * dis
    alpha = 1.0 / (1.0 + jnp.exp(-a_ref[0, 0]))
    m1 = jnp.dot(sh, wl_ref[...], preferred_element_type=jnp.float32)
    m2 = jnp.dot(h_ref[...] - sh, wh_ref[...], preferred_element_type=jnp.float32)
    out_ref[...] = alpha * m1 + (1.0 - alpha) * m2 + b_ref[...]


def kernel(x, edge_index, W_LP1, W_HP1, b1, a1, W_LP2, W_HP2, b2, a2):
    n, f_in = x.shape
    hid = W_LP1.shape[1]
    c_out = W_LP2.shape[1]
    e = edge_index.shape[1]
    nb1 = f_in // LANES
    nb2 = hid // LANES
    rb = 1000
    n_rb = n // rb

    ep_rows = -(-e // (N_SUB * LANES * 2)) * (N_SUB * 2)
    pad = ep_rows * LANES - e
    rows = edge_index[0]
    cols = edge_index[1]
    pad_ix = lax.iota(jnp.int32, pad)
    rowsp = jnp.concatenate([rows, n + (pad_ix % DUMMY)]).reshape(ep_rows, LANES)
    colsp = jnp.concatenate([cols, pad_ix % n]).reshape(ep_rows, LANES)
    zeros_wide = jnp.zeros((n + DUMMY, LANES), jnp.float32)

    degp = _make_deg_kernel(n, ep_rows)(rowsp, zeros_wide)

    dis, xp1 = pl.pallas_call(
        _dis_scale_kernel,
        grid=(n_rb, nb1),
        in_specs=[
            pl.BlockSpec((N_SC, rb, LANES), lambda i, b: (0, i, 0)),
            pl.BlockSpec((rb, LANES), lambda i, b: (i, b)),
        ],
        out_specs=[
            pl.BlockSpec((rb, 1), lambda i, b: (i, 0)),
            pl.BlockSpec((1, rb, LANES), lambda i, b: (b, i, 0)),
        ],
        out_shape=[
            jax.ShapeDtypeStruct((n, 1), jnp.float32),
            jax.ShapeDtypeStruct((nb1, n, LANES), jnp.float32),
        ],
    )(degp, x)

    z1 = _make_spmm_kernel(n, ep_rows, nb1)(
        xp1.reshape(nb1 * n, LANES), rowsp, colsp)

    h, xp2 = pl.pallas_call(
        _layer1_kernel,
        grid=(n_rb,),
        in_specs=[
            pl.BlockSpec((nb1, rb, LANES), lambda i: (0, i, 0)),
            pl.BlockSpec((rb, f_in), lambda i: (i, 0)),
            pl.BlockSpec((rb, 1), lambda i: (i, 0)),
            pl.BlockSpec((f_in, hid), lambda i: (0, 0)),
            pl.BlockSpec((f_in, hid), lambda i: (0, 0)),
            pl.BlockSpec((hid,), lambda i: (0,)),
            pl.BlockSpec((1, 1), lambda i: (0, 0)),
        ],
        out_specs=[
            pl.BlockSpec((rb, hid), lambda i: (i, 0)),
            pl.BlockSpec((nb2, rb, LANES), lambda i: (0, i, 0)),
        ],
        out_shape=[
            jax.ShapeDtypeStruct((n, hid), jnp.float32),
            jax.ShapeDtypeStruct((nb2, n, LANES), jnp.float32),
        ],
    )(z1.reshape(nb1, n, LANES), x, dis, W_LP1, W_HP1, b1, a1.reshape(1, 1))

    z2 = _make_spmm_kernel(n, ep_rows, nb2)(
        xp2.reshape(nb2 * n, LANES), rowsp, colsp)

    out = pl.pallas_call(
        _layer2_kernel,
        grid=(n_rb,),
        in_specs=[
            pl.BlockSpec((nb2, rb, LANES), lambda i: (0, i, 0)),
            pl.BlockSpec((rb, hid), lambda i: (i, 0)),
            pl.BlockSpec((rb, 1), lambda i: (i, 0)),
            pl.BlockSpec((hid, c_out), lambda i: (0, 0)),
            pl.BlockSpec((hid, c_out), lambda i: (0, 0)),
            pl.BlockSpec((c_out,), lambda i: (0,)),
            pl.BlockSpec((1, 1), lambda i: (0, 0)),
        ],
        out_specs=pl.BlockSpec((rb, c_out), lambda i: (i, 0)),
        out_shape=jax.ShapeDtypeStruct((n, c_out), jnp.float32),
    )(z2.reshape(nb2, n, LANES), h, dis, W_LP2, W_HP2, b2, a2.reshape(1, 1))

    return out
